# Initial kernel scaffold; baseline (speedup 1.0000x reference)
import functools

import jax
import jax.numpy as jnp
from jax import lax
from jax.experimental import pallas as pl
from jax.experimental.pallas import tpu as pltpu
from jax.experimental.pallas import tpu_sc as plsc

N = 10000
E = 320000
D = 128
C = 128
NCHUNK = E // C
NC = 2
NS = 16
NW = NC * NS
ROWS_PER_TILE = N // NS
DEGW = 16


def _fill2d(ref, rows, width, value):
    vec = jnp.full((16,), value, dtype=jnp.float32)

    def body(r, _):
        for k in range(width // 16):
            ref[r, pl.ds(16 * k, 16)] = vec
        return 0

    lax.fori_loop(0, rows, body, 0)


def _make_sc_agg(with_deg: bool):
    out_type = [jax.ShapeDtypeStruct((NC, N, D), jnp.float32)]
    scratch = [
        pltpu.VMEM((C,), jnp.int32),
        pltpu.VMEM((C,), jnp.int32),
        pltpu.VMEM((C, D), jnp.float32),
        pltpu.VMEM_SHARED((N, D), jnp.float32),
        pltpu.SemaphoreType.DMA,
    ]
    if with_deg:
        out_type.append(jax.ShapeDtypeStruct((NC, N, DEGW), jnp.float32))
        scratch.append(pltpu.VMEM((C, DEGW), jnp.float32))
        scratch.append(pltpu.VMEM_SHARED((N, DEGW), jnp.float32))

    mesh = plsc.VectorSubcoreMesh(core_axis_name="c", subcore_axis_name="s")

    def body(x_hbm, src_hbm, dst_hbm, *refs):
        if with_deg:
            (agg_hbm, deg_hbm, src_v, dst_v, rows_v, acc_sh, sem,
             ones_v, deg_sh) = refs
        else:
            agg_hbm, src_v, dst_v, rows_v, acc_sh, sem = refs

        cid = lax.axis_index("c")
        sid = lax.axis_index("s")
        wid = cid * NS + sid
        base = sid * ROWS_PER_TILE

        _fill2d(rows_v, C, D, 0.0)
        nfull = ROWS_PER_TILE // C
        rem = ROWS_PER_TILE - nfull * C
        for q in range(nfull):
            pltpu.sync_copy(rows_v, acc_sh.at[pl.ds(base + q * C, C)])
        pltpu.sync_copy(rows_v.at[pl.ds(0, rem)],
                        acc_sh.at[pl.ds(base + nfull * C, rem)])
        if with_deg:
            _fill2d(ones_v, C, DEGW, 0.0)
            for q in range(nfull):
                pltpu.sync_copy(ones_v, deg_sh.at[pl.ds(base + q * C, C)])
            pltpu.sync_copy(ones_v.at[pl.ds(0, rem)],
                            deg_sh.at[pl.ds(base + nfull * C, rem)])
            _fill2d(ones_v, C, DEGW, 1.0)
        plsc.subcore_barrier()

        def chunk_body(j, _):
            c = wid + NW * j

            @pl.when(c < NCHUNK)
            def _():
                pltpu.sync_copy(src_hbm.at[c], src_v)
                pltpu.sync_copy(dst_hbm.at[c], dst_v)
                pltpu.async_copy(x_hbm.at[src_v], rows_v, sem).wait()
                pltpu.sync_copy(rows_v, acc_sh.at[dst_v], add=True)
                if with_deg:
                    pltpu.sync_copy(ones_v, deg_sh.at[dst_v], add=True)

            return 0

        niter = -(-NCHUNK // NW)
        lax.fori_loop(0, niter, chunk_body, 0)
        plsc.subcore_barrier()

        pltpu.sync_copy(acc_sh.at[pl.ds(base, ROWS_PER_TILE)],
                        agg_hbm.at[cid, pl.ds(base, ROWS_PER_TILE)])
        if with_deg:
            pltpu.sync_copy(deg_sh.at[pl.ds(base, ROWS_PER_TILE)],
                            deg_hbm.at[cid, pl.ds(base, ROWS_PER_TILE)])

    out = tuple(out_type) if with_deg else out_type[0]
    return pl.kernel(body, out_type=out, mesh=mesh, scratch_types=scratch)


_sc_agg_deg = _make_sc_agg(True)
_sc_agg = _make_sc_agg(False)

_BLK = 1000


def _dense_body(relu, x_ref, aggp_ref, degp_ref, ws_ref, wn_ref, b_ref, o_ref):
    agg = aggp_ref[0] + aggp_ref[1]
    deg = jnp.sum(degp_ref[0] + degp_ref[1], axis=1, keepdims=True) / DEGW
    aggn = agg / jnp.maximum(deg, 1.0)
    o = (jnp.dot(x_ref[...], ws_ref[...], preferred_element_type=jnp.float32)
         + jnp.dot(aggn, wn_ref[...], preferred_element_type=jnp.float32)
         + b_ref[...])
    o_ref[...] = jnp.maximum(o, 0.0) if relu else o


def _dense(x, aggp, degp, Ws, Wn, b, relu):
    return pl.pallas_call(
        functools.partial(_dense_body, relu),
        grid=(N // _BLK,),
        in_specs=[
            pl.BlockSpec((_BLK, D), lambda i: (i, 0)),
            pl.BlockSpec((NC, _BLK, D), lambda i: (0, i, 0)),
            pl.BlockSpec((NC, _BLK, DEGW), lambda i: (0, i, 0)),
            pl.BlockSpec((D, D), lambda i: (0, 0)),
            pl.BlockSpec((D, D), lambda i: (0, 0)),
            pl.BlockSpec((1, D), lambda i: (0, 0)),
        ],
        out_specs=pl.BlockSpec((_BLK, D), lambda i: (i, 0)),
        out_shape=jax.ShapeDtypeStruct((N, D), jnp.float32),
    )(x, aggp, degp, Ws, Wn, b.reshape(1, D))


def kernel(inputs, edge_index, W_self1, W_neigh1, b1, W_self2, W_neigh2, b2,
           W_self3, W_neigh3, b3):
    src2d = edge_index[0].reshape(NCHUNK, C)
    dst2d = edge_index[1].reshape(NCHUNK, C)

    aggp1, degp = _sc_agg_deg(inputs, src2d, dst2d)
    h1 = _dense(inputs, aggp1, degp, W_self1, W_neigh1, b1, relu=True)
    aggp2 = _sc_agg(h1, src2d, dst2d)
    h2 = _dense(h1, aggp2, degp, W_self2, W_neigh2, b2, relu=True)
    aggp3 = _sc_agg(h2, src2d, dst2d)
    return _dense(h2, aggp3, degp, W_self3, W_neigh3, b3, relu=False)


# speedup vs baseline: 6.5378x; 6.5378x over previous
import functools

import jax
import jax.numpy as jnp
from jax import lax
from jax.experimental import pallas as pl
from jax.experimental.pallas import tpu as pltpu
from jax.experimental.pallas import tpu_sc as plsc

N = 10000
N_PAD = 10240
E = 320000
D = 128
C = 128
NCHUNK = E // C
NC = 2
NS = 16
NW = NC * NS
ROWS_PER_TILE = N_PAD // NS
DEGW = 16


def _fill2d(ref, rows, width, value):
    vec = jnp.full((16,), value, dtype=jnp.float32)

    def body(r, _):
        for k in range(width // 16):
            ref[r, pl.ds(16 * k, 16)] = vec
        return 0

    lax.fori_loop(0, rows, body, 0)


def _make_sc_agg(with_deg: bool):
    out_type = [jax.ShapeDtypeStruct((NC, N_PAD, D), jnp.float32)]
    scratch = [
        pltpu.VMEM((C,), jnp.int32),
        pltpu.VMEM((C,), jnp.int32),
        pltpu.VMEM((C, D), jnp.float32),
        pltpu.VMEM_SHARED((N_PAD, D), jnp.float32),
        pltpu.SemaphoreType.DMA,
    ]
    if with_deg:
        out_type.append(jax.ShapeDtypeStruct((NC, N_PAD, DEGW), jnp.float32))
        scratch.append(pltpu.VMEM((C, DEGW), jnp.float32))
        scratch.append(pltpu.VMEM_SHARED((N_PAD, DEGW), jnp.float32))

    mesh = plsc.VectorSubcoreMesh(core_axis_name="c", subcore_axis_name="s")

    def body(x_hbm, src_hbm, dst_hbm, *refs):
        if with_deg:
            (agg_hbm, deg_hbm, src_v, dst_v, rows_v, acc_sh, sem,
             ones_v, deg_sh) = refs
        else:
            agg_hbm, src_v, dst_v, rows_v, acc_sh, sem = refs

        cid = lax.axis_index("c")
        sid = lax.axis_index("s")
        wid = cid * NS + sid
        base = sid * ROWS_PER_TILE

        _fill2d(rows_v, C, D, 0.0)
        nfull = ROWS_PER_TILE // C
        for q in range(nfull):
            pltpu.sync_copy(rows_v, acc_sh.at[pl.ds(base + q * C, C)])
        if with_deg:
            _fill2d(ones_v, C, DEGW, 0.0)
            for q in range(nfull):
                pltpu.sync_copy(ones_v, deg_sh.at[pl.ds(base + q * C, C)])
            _fill2d(ones_v, C, DEGW, 1.0)
        plsc.subcore_barrier()

        def chunk_body(j, _):
            c = wid + NW * j

            @pl.when(c < NCHUNK)
            def _():
                pltpu.sync_copy(src_hbm.at[c], src_v)
                pltpu.sync_copy(dst_hbm.at[c], dst_v)
                pltpu.async_copy(x_hbm.at[src_v], rows_v, sem).wait()
                pltpu.sync_copy(rows_v, acc_sh.at[dst_v], add=True)
                if with_deg:
                    pltpu.sync_copy(ones_v, deg_sh.at[dst_v], add=True)

            return 0

        niter = -(-NCHUNK // NW)
        lax.fori_loop(0, niter, chunk_body, 0)
        plsc.subcore_barrier()

        pltpu.sync_copy(acc_sh.at[pl.ds(base, ROWS_PER_TILE)],
                        agg_hbm.at[cid, pl.ds(base, ROWS_PER_TILE)])
        if with_deg:
            pltpu.sync_copy(deg_sh.at[pl.ds(base, ROWS_PER_TILE)],
                            deg_hbm.at[cid, pl.ds(base, ROWS_PER_TILE)])

    out = tuple(out_type) if with_deg else out_type[0]
    return pl.kernel(
        body, out_type=out, mesh=mesh, scratch_types=scratch,
        compiler_params=pltpu.CompilerParams(use_tc_tiling_on_sc=False))


_sc_agg_deg = _make_sc_agg(True)
_sc_agg = _make_sc_agg(False)

_BLK = 1000


def _dense_body(relu, x_ref, aggp_ref, degp_ref, ws_ref, wn_ref, b_ref, o_ref):
    agg = aggp_ref[0] + aggp_ref[1]
    deg = jnp.sum(degp_ref[0] + degp_ref[1], axis=1, keepdims=True) / DEGW
    aggn = agg / jnp.maximum(deg, 1.0)
    o = (jnp.dot(x_ref[...], ws_ref[...], preferred_element_type=jnp.float32)
         + jnp.dot(aggn, wn_ref[...], preferred_element_type=jnp.float32)
         + b_ref[...])
    o_ref[...] = jnp.maximum(o, 0.0) if relu else o


def _dense(x, aggp, degp, Ws, Wn, b, relu):
    return pl.pallas_call(
        functools.partial(_dense_body, relu),
        grid=(N // _BLK,),
        in_specs=[
            pl.BlockSpec((_BLK, D), lambda i: (i, 0)),
            pl.BlockSpec((NC, _BLK, D), lambda i: (0, i, 0)),
            pl.BlockSpec((NC, _BLK, DEGW), lambda i: (0, i, 0)),
            pl.BlockSpec((D, D), lambda i: (0, 0)),
            pl.BlockSpec((D, D), lambda i: (0, 0)),
            pl.BlockSpec((1, D), lambda i: (0, 0)),
        ],
        out_specs=pl.BlockSpec((_BLK, D), lambda i: (i, 0)),
        out_shape=jax.ShapeDtypeStruct((N, D), jnp.float32),
    )(x, aggp, degp, Ws, Wn, b.reshape(1, D))


def kernel(inputs, edge_index, W_self1, W_neigh1, b1, W_self2, W_neigh2, b2,
           W_self3, W_neigh3, b3):
    src2d = edge_index[0].reshape(NCHUNK, C)
    dst2d = edge_index[1].reshape(NCHUNK, C)

    aggp1, degp = _sc_agg_deg(inputs, src2d, dst2d)
    h1 = _dense(inputs, aggp1, degp, W_self1, W_neigh1, b1, relu=True)
    aggp2 = _sc_agg(h1, src2d, dst2d)
    h2 = _dense(h1, aggp2, degp, W_self2, W_neigh2, b2, relu=True)
    aggp3 = _sc_agg(h2, src2d, dst2d)
    return _dense(h2, aggp3, degp, W_self3, W_neigh3, b3, relu=False)
